# baseline (speedup 1.0000x reference)
import functools

import jax
import jax.numpy as jnp
from jax import lax
from jax.experimental import pallas as pl
from jax.experimental.pallas import tpu as pltpu
from jax.experimental.pallas import tpu_sc as plsc

NC = 2
NS = 16
L = 16

KOFF = 13
DIM = 8
ZCOLS = 16
GDIM = 32
PAD_ROWS = 1024
CHUNK = 128
CPT = KOFF
NTN = CPT * CHUNK
HALF = NS * NTN
WIN = 1024
UNROLL = 8


def _body(n_nodes, m_pairs, km_hbm, z_hbm, wc_hbm, out_hbm,
          win_sh, win_v, obuf, ibuf, widx, zb, ob, wc_v,
          sem, sem_w, sem_a, sem_b):
    h = lax.axis_index("c")
    s = lax.axis_index("s")
    iota = lax.iota(jnp.int32, L)
    half_lo = h * HALF

    @pl.when(s < KOFF)
    def _():
        k = s
        o_base = k * 2 * m_pairs
        i_base = o_base + m_pairs

        def init_body(j, _):
            for u in range(8):
                node = half_lo + (j * 8 + u) * L + iota
                win_v[pl.ds((j * 8 + u) * L, L)] = n_nodes + (node & (PAD_ROWS - 1))
            return 0

        lax.fori_loop(0, HALF // (L * 8), init_body, 0)

        n_win = m_pairs // WIN

        pltpu.async_copy(km_hbm.at[pl.ds(o_base, WIN)], obuf.at[0], sem).wait()
        pltpu.async_copy(km_hbm.at[pl.ds(i_base, WIN)], ibuf.at[0], sem).wait()

        def win_body(w, _):
            slot = lax.rem(w, 2)
            nslot = lax.rem(w + 1, 2)

            @pl.when(w + 1 < n_win)
            def _():
                pltpu.async_copy(
                    km_hbm.at[pl.ds(o_base + (w + 1) * WIN, WIN)], obuf.at[nslot], sem)
                pltpu.async_copy(
                    km_hbm.at[pl.ds(i_base + (w + 1) * WIN, WIN)], ibuf.at[nslot], sem)

            def pair_body(j, _):
                for u in range(UNROLL):
                    o = obuf[slot, pl.ds((j * UNROLL + u) * L, L)]
                    iv = ibuf[slot, pl.ds((j * UNROLL + u) * L, L)]
                    ol = o - half_lo
                    inhalf = (ol >= 0) & (ol < HALF)
                    plsc.store_scatter(win_v, [ol], iv, mask=inhalf)
                return 0

            lax.fori_loop(0, WIN // (L * UNROLL), pair_body, 0)

            @pl.when(w + 1 < n_win)
            def _():
                pltpu.make_async_copy(
                    km_hbm.at[pl.ds(o_base + (w + 1) * WIN, WIN)], obuf.at[nslot], sem).wait()
                pltpu.make_async_copy(
                    km_hbm.at[pl.ds(i_base + (w + 1) * WIN, WIN)], ibuf.at[nslot], sem).wait()

            return 0

        lax.fori_loop(0, n_win, win_body, 0)
        pltpu.sync_copy(win_v, win_sh.at[k])

    plsc.subcore_barrier()

    base_local = s * NTN
    base_node = half_lo + base_local

    pltpu.sync_copy(wc_hbm, wc_v)
    wc_vecs = [wc_v[pl.ds(16 * q, 16)] for q in range(GDIM * DIM // 16)]
    wc_s = [[wc_vecs[(j * DIM + p) // 16][(j * DIM + p) % 16]
             for p in range(DIM)] for j in range(GDIM)]

    k_splat = [jnp.full((L,), k, jnp.int32) for k in range(KOFF)]
    col_splat = [jnp.full((L,), c, jnp.int32) for c in range(ZCOLS)]
    sems = [sem_a, sem_b]

    def start_gathers(c, slot):
        for k in range(KOFF):
            pltpu.async_copy(
                win_sh.at[k, pl.ds(base_local + c * CHUNK, CHUNK)],
                widx.at[slot, k], sem_w)
        for k in range(KOFF):
            pltpu.make_async_copy(
                win_sh.at[k, pl.ds(base_local + c * CHUNK, CHUNK)],
                widx.at[slot, k], sem_w).wait()
        for k in range(KOFF):
            pltpu.async_copy(
                z_hbm.at[widx.at[slot, k]], zb.at[slot, k], sems[slot])

    def wait_gathers(c, slot):
        for k in range(KOFF):
            pltpu.make_async_copy(
                z_hbm.at[widx.at[slot, k]], zb.at[slot, k], sems[slot]).wait()

    start_gathers(0, 0)

    def chunk_body(c, _):
        slot = lax.rem(c, 2)

        for par in (0, 1):
            @pl.when((slot == par) & (c + 1 < CPT))
            def _(par=par):
                start_gathers(c + 1, 1 - par)

        for par in (0, 1):
            @pl.when(slot == par)
            def _(par=par):
                wait_gathers(c, par)

        slot_splat = jnp.full((L,), slot, jnp.int32)

        def group_body(g, _):
            rows = g * L + iota
            acc = [jnp.zeros((L,), jnp.float32) for _ in range(GDIM)]
            for k in range(KOFF):
                f = [plsc.load_gather(zb, [slot_splat, k_splat[k], rows, col_splat[i]])
                     for i in range(DIM)]
                x = [plsc.load_gather(zb, [slot_splat, k_splat[k], rows, col_splat[DIM + cc]])
                     for cc in range(3)]
                for i in range(DIM):
                    acc[24 + i] = acc[24 + i] + f[i]
                    for cc in range(3):
                        acc[i * 3 + cc] = acc[i * 3 + cc] + f[i] * x[cc]
            orow = g * L + iota
            for p in range(DIM):
                o = acc[0] * wc_s[0][p]
                for j in range(1, GDIM):
                    o = o + acc[j] * wc_s[j][p]
                plsc.store_scatter(ob, [orow, col_splat[p]], o)
            return 0

        lax.fori_loop(0, CHUNK // L, group_body, 0)
        pltpu.sync_copy(ob, out_hbm.at[pl.ds(base_node + c * CHUNK, CHUNK)])
        return 0

    lax.fori_loop(0, CPT, chunk_body, 0)


def kernel(features, coords, kernel_map, W, b):
    n, dim = features.shape
    m_pairs = kernel_map.shape[2]
    n_pad_out = NC * HALF

    m_pad = ((m_pairs + WIN - 1) // WIN) * WIN
    if m_pad != m_pairs:
        pad_o = jnp.full((KOFF, 1, m_pad - m_pairs), n_pad_out - 1, jnp.int32)
        pad_i = jnp.zeros((KOFF, 1, m_pad - m_pairs), jnp.int32)
        km = jnp.concatenate(
            [kernel_map, jnp.concatenate([pad_o, pad_i], axis=1)], axis=2)
    else:
        km = kernel_map

    z = jnp.concatenate(
        [features, coords, jnp.zeros((n, ZCOLS - dim - 3), jnp.float32)], axis=1)
    z = jnp.concatenate([z, jnp.zeros((PAD_ROWS, ZCOLS), jnp.float32)], axis=0)

    wc_top = jnp.transpose(W.reshape(DIM, DIM, 3), (0, 2, 1)).reshape(3 * DIM, DIM)
    wc = jnp.concatenate([wc_top, b.reshape(DIM, DIM)], axis=0).reshape(-1)

    mesh = plsc.VectorSubcoreMesh(core_axis_name="c", subcore_axis_name="s")
    params = pltpu.CompilerParams(
        needs_layout_passes=False, use_tc_tiling_on_sc=False)

    fused = pl.kernel(
        functools.partial(_body, n, m_pad),
        out_type=jax.ShapeDtypeStruct((n_pad_out, DIM), jnp.float32),
        mesh=mesh,
        compiler_params=params,
        scratch_types=[
            pltpu.VMEM_SHARED((KOFF, HALF), jnp.int32),
            pltpu.VMEM((HALF,), jnp.int32),
            pltpu.VMEM((2, WIN), jnp.int32),
            pltpu.VMEM((2, WIN), jnp.int32),
            pltpu.VMEM((2, KOFF, CHUNK), jnp.int32),
            pltpu.VMEM((2, KOFF, CHUNK, ZCOLS), jnp.float32),
            pltpu.VMEM((CHUNK, DIM), jnp.float32),
            pltpu.VMEM((GDIM * DIM,), jnp.float32),
            pltpu.SemaphoreType.DMA,
            pltpu.SemaphoreType.DMA,
            pltpu.SemaphoreType.DMA,
            pltpu.SemaphoreType.DMA,
        ],
    )
    out = fused(km.reshape(-1), z, wc)
    return out[:n]


# speedup vs baseline: 1.0680x; 1.0680x over previous
import functools

import jax
import jax.numpy as jnp
from jax import lax
from jax.experimental import pallas as pl
from jax.experimental.pallas import tpu as pltpu
from jax.experimental.pallas import tpu_sc as plsc

NC = 2
NS = 16
L = 16

KOFF = 13
DIM = 8
ZCOLS = 16
GDIM = 32
PAD_ROWS = 1024
CHUNK = 128
CPT = KOFF
NTN = CPT * CHUNK
HALF = NS * NTN
WIN = 1024
UNROLL = 8


def _body(n_nodes, m_pairs, km_hbm, z_hbm, wc_hbm, out_hbm,
          win_sh, win_v, obuf, ibuf, widx, zb, ob, wc_v,
          sem, sem_w, sem_a, sem_b):
    h = lax.axis_index("c")
    s = lax.axis_index("s")
    iota = lax.iota(jnp.int32, L)
    half_lo = h * HALF

    @pl.when(s < KOFF)
    def _():
        k = s
        o_base = k * 2 * m_pairs
        i_base = o_base + m_pairs

        def init_body(j, _):
            for u in range(8):
                node = half_lo + (j * 8 + u) * L + iota
                win_v[pl.ds((j * 8 + u) * L, L)] = n_nodes + (node & (PAD_ROWS - 1))
            return 0

        lax.fori_loop(0, HALF // (L * 8), init_body, 0)

        n_win = m_pairs // WIN
        tail = m_pairs - n_win * WIN
        tail_dma = ((tail + 127) // 128) * 128
        tail_start = m_pairs - tail_dma
        skip_vregs = (tail_dma - tail) // L

        pltpu.async_copy(km_hbm.at[pl.ds(o_base, WIN)], obuf.at[0], sem).wait()
        pltpu.async_copy(km_hbm.at[pl.ds(i_base, WIN)], ibuf.at[0], sem).wait()

        def win_body(w, _):
            slot = lax.rem(w, 2)
            nslot = lax.rem(w + 1, 2)

            @pl.when(w + 1 < n_win)
            def _():
                pltpu.async_copy(
                    km_hbm.at[pl.ds(o_base + (w + 1) * WIN, WIN)], obuf.at[nslot], sem)
                pltpu.async_copy(
                    km_hbm.at[pl.ds(i_base + (w + 1) * WIN, WIN)], ibuf.at[nslot], sem)

            def pair_body(j, _):
                for u in range(UNROLL):
                    o = obuf[slot, pl.ds((j * UNROLL + u) * L, L)]
                    iv = ibuf[slot, pl.ds((j * UNROLL + u) * L, L)]
                    ol = o - half_lo
                    inhalf = (ol >= 0) & (ol < HALF)
                    plsc.store_scatter(win_v, [ol], iv, mask=inhalf)
                return 0

            lax.fori_loop(0, WIN // (L * UNROLL), pair_body, 0)

            @pl.when(w + 1 < n_win)
            def _():
                pltpu.make_async_copy(
                    km_hbm.at[pl.ds(o_base + (w + 1) * WIN, WIN)], obuf.at[nslot], sem).wait()
                pltpu.make_async_copy(
                    km_hbm.at[pl.ds(i_base + (w + 1) * WIN, WIN)], ibuf.at[nslot], sem).wait()

            return 0

        lax.fori_loop(0, n_win, win_body, 0)

        if tail > 0:
            pltpu.async_copy(
                km_hbm.at[pl.ds(o_base + tail_start, tail_dma)],
                obuf.at[0, pl.ds(0, tail_dma)], sem).wait()
            pltpu.async_copy(
                km_hbm.at[pl.ds(i_base + tail_start, tail_dma)],
                ibuf.at[0, pl.ds(0, tail_dma)], sem).wait()

            def tail_body(j, _):
                o = obuf[0, pl.ds(j * L, L)]
                iv = ibuf[0, pl.ds(j * L, L)]
                ol = o - half_lo
                inhalf = (ol >= 0) & (ol < HALF)
                plsc.store_scatter(win_v, [ol], iv, mask=inhalf)
                return 0

            lax.fori_loop(skip_vregs, tail_dma // L, tail_body, 0)

        pltpu.sync_copy(win_v, win_sh.at[k])

    plsc.subcore_barrier()

    base_local = s * NTN
    base_node = half_lo + base_local

    pltpu.sync_copy(wc_hbm, wc_v)
    wc_vecs = [wc_v[pl.ds(16 * q, 16)] for q in range(GDIM * DIM // 16)]
    wc_s = [[wc_vecs[(j * DIM + p) // 16][(j * DIM + p) % 16]
             for p in range(DIM)] for j in range(GDIM)]

    k_splat = [jnp.full((L,), k, jnp.int32) for k in range(KOFF)]
    col_splat = [jnp.full((L,), c, jnp.int32) for c in range(ZCOLS)]
    sems = [sem_a, sem_b]

    def start_gathers(c, slot):
        for k in range(KOFF):
            pltpu.async_copy(
                win_sh.at[k, pl.ds(base_local + c * CHUNK, CHUNK)],
                widx.at[slot, k], sem_w)
        for k in range(KOFF):
            pltpu.make_async_copy(
                win_sh.at[k, pl.ds(base_local + c * CHUNK, CHUNK)],
                widx.at[slot, k], sem_w).wait()
        for k in range(KOFF):
            pltpu.async_copy(
                z_hbm.at[widx.at[slot, k]], zb.at[slot, k], sems[slot])

    def wait_gathers(c, slot):
        for k in range(KOFF):
            pltpu.make_async_copy(
                z_hbm.at[widx.at[slot, k]], zb.at[slot, k], sems[slot]).wait()

    start_gathers(0, 0)

    def chunk_body(c, _):
        slot = lax.rem(c, 2)

        for par in (0, 1):
            @pl.when((slot == par) & (c + 1 < CPT))
            def _(par=par):
                start_gathers(c + 1, 1 - par)

        for par in (0, 1):
            @pl.when(slot == par)
            def _(par=par):
                wait_gathers(c, par)

        slot_splat = jnp.full((L,), slot, jnp.int32)

        def group_body(g, _):
            rows = g * L + iota
            acc = [jnp.zeros((L,), jnp.float32) for _ in range(GDIM)]
            for k in range(KOFF):
                f = [plsc.load_gather(zb, [slot_splat, k_splat[k], rows, col_splat[i]])
                     for i in range(DIM)]
                x = [plsc.load_gather(zb, [slot_splat, k_splat[k], rows, col_splat[DIM + cc]])
                     for cc in range(3)]
                for i in range(DIM):
                    acc[24 + i] = acc[24 + i] + f[i]
                    for cc in range(3):
                        acc[i * 3 + cc] = acc[i * 3 + cc] + f[i] * x[cc]
            orow = g * L + iota
            for p in range(DIM):
                o = acc[0] * wc_s[0][p]
                for j in range(1, GDIM):
                    o = o + acc[j] * wc_s[j][p]
                plsc.store_scatter(ob, [orow, col_splat[p]], o)
            return 0

        lax.fori_loop(0, CHUNK // L, group_body, 0)

        g0 = base_node + c * CHUNK

        @pl.when(g0 + CHUNK <= n_nodes)
        def _():
            pltpu.sync_copy(ob, out_hbm.at[pl.ds(g0, CHUNK)])

        part = n_nodes % CHUNK
        if part:
            @pl.when((g0 < n_nodes) & (g0 + CHUNK > n_nodes))
            def _():
                pltpu.sync_copy(
                    ob.at[pl.ds(0, part)], out_hbm.at[pl.ds(g0, part)])
        return 0

    lax.fori_loop(0, CPT, chunk_body, 0)


def kernel(features, coords, kernel_map, W, b):
    n, dim = features.shape
    m_pairs = kernel_map.shape[2]

    z = jnp.concatenate(
        [features, coords, jnp.zeros((n, ZCOLS - dim - 3), jnp.float32)], axis=1)
    z = jnp.concatenate([z, jnp.zeros((PAD_ROWS, ZCOLS), jnp.float32)], axis=0)

    wc_top = jnp.transpose(W.reshape(DIM, DIM, 3), (0, 2, 1)).reshape(3 * DIM, DIM)
    wc = jnp.concatenate([wc_top, b.reshape(DIM, DIM)], axis=0).reshape(-1)

    mesh = plsc.VectorSubcoreMesh(core_axis_name="c", subcore_axis_name="s")
    params = pltpu.CompilerParams(
        needs_layout_passes=False, use_tc_tiling_on_sc=False)

    fused = pl.kernel(
        functools.partial(_body, n, m_pairs),
        out_type=jax.ShapeDtypeStruct((n, DIM), jnp.float32),
        mesh=mesh,
        compiler_params=params,
        scratch_types=[
            pltpu.VMEM_SHARED((KOFF, HALF), jnp.int32),
            pltpu.VMEM((HALF,), jnp.int32),
            pltpu.VMEM((2, WIN), jnp.int32),
            pltpu.VMEM((2, WIN), jnp.int32),
            pltpu.VMEM((2, KOFF, CHUNK), jnp.int32),
            pltpu.VMEM((2, KOFF, CHUNK, ZCOLS), jnp.float32),
            pltpu.VMEM((CHUNK, DIM), jnp.float32),
            pltpu.VMEM((GDIM * DIM,), jnp.float32),
            pltpu.SemaphoreType.DMA,
            pltpu.SemaphoreType.DMA,
            pltpu.SemaphoreType.DMA,
            pltpu.SemaphoreType.DMA,
        ],
    )
    return fused(kernel_map.reshape(-1), z, wc)


# speedup vs baseline: 1.2473x; 1.1678x over previous
import functools

import jax
import jax.numpy as jnp
from jax import lax
from jax.experimental import pallas as pl
from jax.experimental.pallas import tpu as pltpu
from jax.experimental.pallas import tpu_sc as plsc

NC = 2
NS = 16
NW = NC * NS
L = 16

KOFF = 13
DIM = 8
ZCOLS = 16
GDIM = 32
PAD_ROWS = 1024
CHUNK = 128
CPT = KOFF
NTN = CPT * CHUNK
WIN = 2048
UNROLL = 8


def _winner_body(n_nodes, n_pad_out, m_pairs, km_hbm, win_hbm, win_v, obuf, ibuf, sem):
    wid = lax.axis_index("s") * NC + lax.axis_index("c")

    @pl.when(wid < KOFF)
    def _():
        k = wid
        iota = lax.iota(jnp.int32, L)
        o_base = k * 2 * m_pairs
        i_base = o_base + m_pairs

        def init_body(j, _):
            for u in range(8):
                node = (j * 8 + u) * L + iota
                win_v[pl.ds((j * 8 + u) * L, L)] = n_nodes + (node & (PAD_ROWS - 1))
            return 0

        lax.fori_loop(0, n_pad_out // (L * 8), init_body, 0)

        n_win = m_pairs // WIN

        pltpu.async_copy(km_hbm.at[pl.ds(o_base, WIN)], obuf.at[0], sem).wait()
        pltpu.async_copy(km_hbm.at[pl.ds(i_base, WIN)], ibuf.at[0], sem).wait()

        def win_body(w, _):
            slot = lax.rem(w, 2)
            nslot = lax.rem(w + 1, 2)

            @pl.when(w + 1 < n_win)
            def _():
                pltpu.async_copy(
                    km_hbm.at[pl.ds(o_base + (w + 1) * WIN, WIN)], obuf.at[nslot], sem)
                pltpu.async_copy(
                    km_hbm.at[pl.ds(i_base + (w + 1) * WIN, WIN)], ibuf.at[nslot], sem)

            def pair_body(j, _):
                for u in range(UNROLL):
                    o = obuf[slot, pl.ds((j * UNROLL + u) * L, L)]
                    iv = ibuf[slot, pl.ds((j * UNROLL + u) * L, L)]
                    plsc.store_scatter(win_v, [o], iv)
                return 0

            lax.fori_loop(0, WIN // (L * UNROLL), pair_body, 0)

            @pl.when(w + 1 < n_win)
            def _():
                pltpu.make_async_copy(
                    km_hbm.at[pl.ds(o_base + (w + 1) * WIN, WIN)], obuf.at[nslot], sem).wait()
                pltpu.make_async_copy(
                    km_hbm.at[pl.ds(i_base + (w + 1) * WIN, WIN)], ibuf.at[nslot], sem).wait()

            return 0

        lax.fori_loop(0, n_win, win_body, 0)

        tail = m_pairs - n_win * WIN
        if tail > 0:
            tail_dma = ((tail + 127) // 128) * 128
            tail_start = m_pairs - tail_dma
            skip_vregs = (tail_dma - tail) // L
            pltpu.async_copy(
                km_hbm.at[pl.ds(o_base + tail_start, tail_dma)],
                obuf.at[0, pl.ds(0, tail_dma)], sem).wait()
            pltpu.async_copy(
                km_hbm.at[pl.ds(i_base + tail_start, tail_dma)],
                ibuf.at[0, pl.ds(0, tail_dma)], sem).wait()

            def tail_body(j, _):
                o = obuf[0, pl.ds(j * L, L)]
                iv = ibuf[0, pl.ds(j * L, L)]
                plsc.store_scatter(win_v, [o], iv)
                return 0

            lax.fori_loop(skip_vregs, tail_dma // L, tail_body, 0)

        pltpu.sync_copy(win_v, win_hbm.at[pl.ds(k * n_pad_out, n_pad_out)])


def _accum_body(n_nodes, n_pad_out, z_hbm, wc_hbm, win_hbm, out_hbm,
                wc_v, win_t, zb, ob, sem_w, sem_a, sem_b):
    wid = lax.axis_index("s") * NC + lax.axis_index("c")
    iota = lax.iota(jnp.int32, L)
    base_node = wid * NTN

    wcopies = [
        pltpu.async_copy(
            win_hbm.at[pl.ds(k * n_pad_out + base_node, NTN)], win_t.at[k], sem_w)
        for k in range(KOFF)
    ]
    pltpu.sync_copy(wc_hbm, wc_v)
    wc_vecs = [wc_v[pl.ds(16 * q, 16)] for q in range(GDIM * DIM // 16)]
    wc_s = [[wc_vecs[(j * DIM + p) // 16][(j * DIM + p) % 16]
             for p in range(DIM)] for j in range(GDIM)]
    for cp in wcopies:
        cp.wait()

    k_splat = [jnp.full((L,), k, jnp.int32) for k in range(KOFF)]
    col_splat = [jnp.full((L,), c, jnp.int32) for c in range(ZCOLS)]
    sems = [sem_a, sem_b]

    def start_gathers(c, slot):
        for k in range(KOFF):
            pltpu.async_copy(
                z_hbm.at[win_t.at[k, pl.ds(c * CHUNK, CHUNK)]],
                zb.at[slot, k], sems[slot])

    def wait_gathers(c, slot):
        for k in range(KOFF):
            pltpu.make_async_copy(
                z_hbm.at[win_t.at[k, pl.ds(c * CHUNK, CHUNK)]],
                zb.at[slot, k], sems[slot]).wait()

    start_gathers(0, 0)

    def chunk_body(c, _):
        slot = lax.rem(c, 2)

        for par in (0, 1):
            @pl.when((slot == par) & (c + 1 < CPT))
            def _(par=par):
                start_gathers(c + 1, 1 - par)

        for par in (0, 1):
            @pl.when(slot == par)
            def _(par=par):
                wait_gathers(c, par)

        slot_splat = jnp.full((L,), slot, jnp.int32)

        def group_body(g, _):
            rows = g * L + iota
            acc = [jnp.zeros((L,), jnp.float32) for _ in range(GDIM)]
            for k in range(KOFF):
                f = [plsc.load_gather(zb, [slot_splat, k_splat[k], rows, col_splat[i]])
                     for i in range(DIM)]
                x = [plsc.load_gather(zb, [slot_splat, k_splat[k], rows, col_splat[DIM + cc]])
                     for cc in range(3)]
                for i in range(DIM):
                    acc[24 + i] = acc[24 + i] + f[i]
                    for cc in range(3):
                        acc[i * 3 + cc] = acc[i * 3 + cc] + f[i] * x[cc]
            orow = c * CHUNK + g * L + iota
            for p in range(DIM):
                o = acc[0] * wc_s[0][p]
                for j in range(1, GDIM):
                    o = o + acc[j] * wc_s[j][p]
                plsc.store_scatter(ob, [orow, col_splat[p]], o)
            return 0

        lax.fori_loop(0, CHUNK // L, group_body, 0)
        return 0

    lax.fori_loop(0, CPT, chunk_body, 0)

    @pl.when(base_node + NTN <= n_nodes)
    def _():
        pltpu.sync_copy(ob, out_hbm.at[pl.ds(base_node, NTN)])

    part = n_nodes % NTN
    if part:
        @pl.when((base_node < n_nodes) & (base_node + NTN > n_nodes))
        def _():
            pltpu.sync_copy(
                ob.at[pl.ds(0, part)], out_hbm.at[pl.ds(base_node, part)])


def kernel(features, coords, kernel_map, W, b):
    n, dim = features.shape
    m_pairs = kernel_map.shape[2]
    n_pad_out = NW * NTN

    z = jnp.concatenate(
        [features, coords, jnp.zeros((n, ZCOLS - dim - 3), jnp.float32)], axis=1)
    z = jnp.concatenate([z, jnp.zeros((PAD_ROWS, ZCOLS), jnp.float32)], axis=0)

    wc_top = jnp.transpose(W.reshape(DIM, DIM, 3), (0, 2, 1)).reshape(3 * DIM, DIM)
    wc = jnp.concatenate([wc_top, b.reshape(DIM, DIM)], axis=0).reshape(-1)

    mesh = plsc.VectorSubcoreMesh(core_axis_name="c", subcore_axis_name="s")
    params = pltpu.CompilerParams(
        needs_layout_passes=False, use_tc_tiling_on_sc=False)

    winner = pl.kernel(
        functools.partial(_winner_body, n, n_pad_out, m_pairs),
        out_type=jax.ShapeDtypeStruct((KOFF * n_pad_out,), jnp.int32),
        mesh=mesh,
        compiler_params=params,
        scratch_types=[
            pltpu.VMEM((n_pad_out,), jnp.int32),
            pltpu.VMEM((2, WIN), jnp.int32),
            pltpu.VMEM((2, WIN), jnp.int32),
            pltpu.SemaphoreType.DMA,
        ],
    )
    win = winner(kernel_map.reshape(-1))

    accum = pl.kernel(
        functools.partial(_accum_body, n, n_pad_out),
        out_type=jax.ShapeDtypeStruct((n, DIM), jnp.float32),
        mesh=mesh,
        compiler_params=params,
        scratch_types=[
            pltpu.VMEM((GDIM * DIM,), jnp.float32),
            pltpu.VMEM((KOFF, NTN), jnp.int32),
            pltpu.VMEM((2, KOFF, CHUNK, ZCOLS), jnp.float32),
            pltpu.VMEM((NTN, DIM), jnp.float32),
            pltpu.SemaphoreType.DMA,
            pltpu.SemaphoreType.DMA,
            pltpu.SemaphoreType.DMA,
        ],
    )
    return accum(z, wc, win)
